# baseline (speedup 1.0000x reference)
import jax
import jax.numpy as jnp
from jax import lax
from jax.experimental import pallas as pl
from jax.experimental.pallas import tpu as pltpu
from jax.experimental.pallas import tpu_sc as plsc

_N = 10000
_D = 128
_E = 320000
_EPS = 1e-5

_NC = 2
_NS = 16
_L = 16
_NW = _NC * _NS
_C = 128
_NCH = 80
_EPW = _NCH * _C
_EPAD = _NW * _EPW
_RPT = 624


def _sc_agg_body(x_hbm, idx_hbm, ew_hbm, out_hbm,
                 acc, idx_a, idx_b, ew_a, ew_b, buf_a, buf_b, zbuf,
                 sem_a, sem_b, sem_i):
    cid = lax.axis_index("c")
    sid = lax.axis_index("s")
    wid = cid * _NS + sid

    def _stage(k, idx, ewc, sync=False):
        if sync:
            pltpu.sync_copy(idx_hbm.at[wid, k], idx)
            pltpu.sync_copy(ew_hbm.at[wid, k], ewc)
        else:
            pltpu.async_copy(idx_hbm.at[wid, k], idx, sem_i)
            pltpu.async_copy(ew_hbm.at[wid, k], ewc, sem_i)

    def _wait_stage(idx, ewc):
        pltpu.make_async_copy(idx_hbm.at[0, 0], idx, sem_i).wait()
        pltpu.make_async_copy(ew_hbm.at[0, 0], ewc, sem_i).wait()

    def _gather(k, idx, buf, sem):
        pltpu.async_copy(x_hbm.at[idx.at[0]], buf, sem)

    def _wait_gather(buf, sem):
        pltpu.make_async_copy(x_hbm.at[pl.ds(0, _C)], buf, sem).wait()

    def _scale(ewc, buf):
        def _grp(g, c2):
            wv16 = ewc[pl.ds(g * _L, _L)]
            for l in range(_L):
                wv = jnp.full((_L,), wv16[l], jnp.float32)
                r = g * _L + l
                for j in range(_D // _L):
                    sl = pl.ds(j * _L, _L)
                    buf[r, sl] = buf[r, sl] * wv
            return c2
        lax.fori_loop(0, _C // _L, _grp, 0)

    def _scatter(idx, buf):
        pltpu.sync_copy(buf, acc.at[idx.at[1]], add=True)

    _stage(0, idx_a, ew_a, sync=True)
    _gather(0, idx_a, buf_a, sem_a)
    _stage(1, idx_b, ew_b)

    def _zrow(i, carry):
        for j in range(_D // _L):
            zbuf[i, pl.ds(j * _L, _L)] = jnp.zeros((_L,), jnp.float32)
        return carry
    lax.fori_loop(0, 16, _zrow, 0)
    row0 = sid * _RPT
    def _zcopy(k, carry):
        pltpu.sync_copy(zbuf, acc.at[pl.ds(row0 + k * 16, 16)])
        return carry
    lax.fori_loop(0, _RPT // 16, _zcopy, 0)
    @pl.when(sid == _NS - 1)
    def _():
        pltpu.sync_copy(zbuf, acc.at[pl.ds(_N - 16, 16)])

    plsc.subcore_barrier()

    def _pair(m, carry):
        _wait_stage(idx_b, ew_b)
        _gather(2 * m + 1, idx_b, buf_b, sem_b)
        _wait_gather(buf_a, sem_a)
        _scale(ew_a, buf_a)
        _scatter(idx_a, buf_a)
        _stage(2 * m + 2, idx_a, ew_a)
        _wait_stage(idx_a, ew_a)
        _gather(2 * m + 2, idx_a, buf_a, sem_a)
        _wait_gather(buf_b, sem_b)
        _scale(ew_b, buf_b)
        _scatter(idx_b, buf_b)
        _stage(2 * m + 3, idx_b, ew_b)
        return carry
    lax.fori_loop(0, _NCH // 2 - 1, _pair, 0)

    _wait_stage(idx_b, ew_b)
    _gather(_NCH - 1, idx_b, buf_b, sem_b)
    _wait_gather(buf_a, sem_a)
    _scale(ew_a, buf_a)
    _scatter(idx_a, buf_a)
    _wait_gather(buf_b, sem_b)
    _scale(ew_b, buf_b)
    _scatter(idx_b, buf_b)

    plsc.subcore_barrier()

    pltpu.sync_copy(acc.at[pl.ds(row0, _RPT)],
                    out_hbm.at[cid, pl.ds(row0, _RPT)])

    @pl.when(sid == _NS - 1)
    def _():
        pltpu.sync_copy(acc.at[pl.ds(_N - 16, 16)],
                        out_hbm.at[cid, pl.ds(_N - 16, 16)])


def _sc_aggregate(x, src, dst, ew):
    pad = _EPAD - _E
    srcp = jnp.concatenate([src, jnp.zeros((pad,), jnp.int32)])
    dstp = jnp.concatenate([dst, jnp.zeros((pad,), jnp.int32)])
    ewp = jnp.concatenate([ew, jnp.zeros((pad,), jnp.float32)])
    packed = jnp.stack([
        srcp.reshape(_NW, _NCH, _C),
        dstp.reshape(_NW, _NCH, _C),
    ], axis=2)
    ewp = ewp.reshape(_NW, _NCH, _C)

    mesh = plsc.VectorSubcoreMesh(core_axis_name="c", subcore_axis_name="s")
    f = pl.kernel(
        _sc_agg_body,
        mesh=mesh,
        out_type=jax.ShapeDtypeStruct((_NC, _N, _D), jnp.float32),
        scratch_types=[
            pltpu.VMEM_SHARED((_N, _D), jnp.float32),
            pltpu.VMEM((2, _C), jnp.int32),
            pltpu.VMEM((2, _C), jnp.int32),
            pltpu.VMEM((_C,), jnp.float32),
            pltpu.VMEM((_C,), jnp.float32),
            pltpu.VMEM((_C, _D), jnp.float32),
            pltpu.VMEM((_C, _D), jnp.float32),
            pltpu.VMEM((16, _D), jnp.float32),
            pltpu.SemaphoreType.DMA,
            pltpu.SemaphoreType.DMA,
            pltpu.SemaphoreType.DMA,
        ],
    )
    return f(x, packed, ewp)


def _tc_tail_body(p_ref, w_ref, b_ref, a1_ref, g_ref, be_ref, a2_ref, o_ref):
    agg = p_ref[0] + p_ref[1]
    h = lax.dot_general(agg, w_ref[...], (((1,), (0,)), ((), ())),
                        preferred_element_type=jnp.float32,
                        precision=lax.Precision.HIGHEST)
    h = h + b_ref[...]
    a1 = a1_ref[0, 0]
    h = jnp.maximum(h, 0.0) + a1 * jnp.minimum(h, 0.0)
    mean = jnp.mean(h, axis=0, keepdims=True)
    var = jnp.mean((h - mean) ** 2, axis=0, keepdims=True)
    h = (h - mean) / jnp.sqrt(var + _EPS) * g_ref[...] + be_ref[...]
    a2 = a2_ref[0, 0]
    o_ref[...] = jnp.maximum(h, 0.0) + a2 * jnp.minimum(h, 0.0)


def _tc_tail(partials, W, b, a1, gamma, beta, a2):
    return pl.pallas_call(
        _tc_tail_body,
        out_shape=jax.ShapeDtypeStruct((_N, _D), jnp.float32),
        in_specs=[
            pl.BlockSpec(memory_space=pltpu.VMEM),
            pl.BlockSpec(memory_space=pltpu.VMEM),
            pl.BlockSpec(memory_space=pltpu.VMEM),
            pl.BlockSpec(memory_space=pltpu.SMEM),
            pl.BlockSpec(memory_space=pltpu.VMEM),
            pl.BlockSpec(memory_space=pltpu.VMEM),
            pl.BlockSpec(memory_space=pltpu.SMEM),
        ],
        out_specs=pl.BlockSpec(memory_space=pltpu.VMEM),
    )(partials, W, b.reshape(1, _D), a1.reshape(1, 1),
      gamma.reshape(1, _D), beta.reshape(1, _D), a2.reshape(1, 1))


def kernel(heat, edge_index, edge_weight, W, b, a1, gamma, beta, a2):
    src = edge_index[0]
    dst = edge_index[1]
    partials = _sc_aggregate(heat, src, dst, edge_weight)
    return _tc_tail(partials, W, b, a1, gamma, beta, a2)


# speedup vs baseline: 3.1463x; 3.1463x over previous
import jax
import jax.numpy as jnp
from jax import lax
from jax.experimental import pallas as pl
from jax.experimental.pallas import tpu as pltpu
from jax.experimental.pallas import tpu_sc as plsc

_N = 10000
_D = 128
_E = 320000
_EPS = 1e-5

_NC = 2
_NS = 16
_L = 16
_NW = _NC * _NS
_C = 128
_NCH = 80
_EPW = _NCH * _C
_EPAD = _NW * _EPW
_RPT = 624


def _sc_agg_body(x_hbm, idx_hbm, ew_hbm, out_hbm,
                 acc, idx0, idx1, idx2, ew0, ew1, ew2, buf0, buf1, buf2,
                 sem_g, sem_s, sem_i):
    cid = lax.axis_index("c")
    sid = lax.axis_index("s")
    wid = cid * _NS + sid
    idxs = (idx0, idx1, idx2)
    ews = (ew0, ew1, ew2)
    bufs = (buf0, buf1, buf2)

    def _stage(k, i, sync=False):
        if sync:
            pltpu.sync_copy(idx_hbm.at[wid, k], idxs[i])
            pltpu.sync_copy(ew_hbm.at[wid, k], ews[i])
        else:
            pltpu.async_copy(idx_hbm.at[wid, k], idxs[i], sem_i)
            pltpu.async_copy(ew_hbm.at[wid, k], ews[i], sem_i)

    def _wait_stage(i):
        pltpu.make_async_copy(idx_hbm.at[0, 0], idxs[i], sem_i).wait()
        pltpu.make_async_copy(ew_hbm.at[0, 0], ews[i], sem_i).wait()

    def _gather(k, i):
        pltpu.async_copy(x_hbm.at[idxs[i].at[0]], bufs[i], sem_g)

    def _wait_gather(i):
        pltpu.make_async_copy(x_hbm.at[pl.ds(0, _C)], bufs[i], sem_g).wait()

    def _scale(i):
        ewc, buf = ews[i], bufs[i]
        def _grp(g, c2):
            wv16 = ewc[pl.ds(g * _L, _L)]
            for l in range(_L):
                wv = jnp.full((_L,), wv16[l], jnp.float32)
                r = g * _L + l
                for j in range(_D // _L):
                    sl = pl.ds(j * _L, _L)
                    buf[r, sl] = buf[r, sl] * wv
            return c2
        lax.fori_loop(0, _C // _L, _grp, 0)

    def _scatter(i):
        pltpu.async_copy(bufs[i], acc.at[idxs[i].at[1]], sem_s, add=True)

    def _wait_scatter(i):
        pltpu.make_async_copy(x_hbm.at[pl.ds(0, _C)], bufs[i], sem_s).wait()

    _stage(0, 0, sync=True)
    _gather(0, 0)
    _stage(1, 1)
    _wait_stage(1)
    _gather(1, 1)

    def _zrow(i, carry):
        for j in range(_D // _L):
            buf2[i, pl.ds(j * _L, _L)] = jnp.zeros((_L,), jnp.float32)
        return carry
    lax.fori_loop(0, _C, _zrow, 0)
    row0 = sid * _RPT
    for k in range(_RPT // _C):
        pltpu.sync_copy(buf2, acc.at[pl.ds(row0 + k * _C, _C)])
    rem = _RPT % _C
    pltpu.sync_copy(buf2.at[pl.ds(0, rem)],
                    acc.at[pl.ds(row0 + _RPT - rem, rem)])
    @pl.when(sid == _NS - 1)
    def _():
        pltpu.sync_copy(buf2.at[pl.ds(0, 16)], acc.at[pl.ds(_N - 16, 16)])

    plsc.subcore_barrier()

    def _step(k, i, first=False, last=False):
        _wait_gather(i)
        _scale(i)
        if not first:
            _wait_scatter((i + 2) % 3)
        if not last:
            _stage(k + 2, (i + 2) % 3)
        _scatter(i)
        if not last:
            _wait_stage((i + 2) % 3)
            _gather(k + 2, (i + 2) % 3)

    _step(0, 0, first=True)

    def _triple(m, carry):
        k = 3 * m + 1
        _step(k, 1)
        _step(k + 1, 2)
        _step(k + 2, 0)
        return carry
    lax.fori_loop(0, (_NCH - 5) // 3, _triple, 0)

    _step(_NCH - 4, (_NCH - 4) % 3)
    _step(_NCH - 3, (_NCH - 3) % 3)
    _step(_NCH - 2, (_NCH - 2) % 3, last=True)
    _step(_NCH - 1, (_NCH - 1) % 3, last=True)
    _wait_scatter((_NCH - 1) % 3)

    plsc.subcore_barrier()

    pltpu.sync_copy(acc.at[pl.ds(row0, _RPT)],
                    out_hbm.at[cid, pl.ds(row0, _RPT)])

    @pl.when(sid == _NS - 1)
    def _():
        pltpu.sync_copy(acc.at[pl.ds(_N - 16, 16)],
                        out_hbm.at[cid, pl.ds(_N - 16, 16)])


def _sc_aggregate(x, src, dst, ew):
    ppw = (_EPAD - _E) // _NW
    rpw = _E // _NW
    lane = jnp.arange(_NW, dtype=jnp.int32)[:, None]
    j = jnp.arange(ppw, dtype=jnp.int32)[None, :]
    fill = (lane * 997 + j * 41) % _N
    srcw = jnp.concatenate([src.reshape(_NW, rpw), fill], axis=1)
    dstw = jnp.concatenate([dst.reshape(_NW, rpw), fill], axis=1)
    eww = jnp.concatenate(
        [ew.reshape(_NW, rpw), jnp.zeros((_NW, ppw), jnp.float32)], axis=1)
    packed = jnp.stack([
        srcw.reshape(_NW, _NCH, _C),
        dstw.reshape(_NW, _NCH, _C),
    ], axis=2)
    ewp = eww.reshape(_NW, _NCH, _C)

    mesh = plsc.VectorSubcoreMesh(core_axis_name="c", subcore_axis_name="s")
    f = pl.kernel(
        _sc_agg_body,
        mesh=mesh,
        out_type=jax.ShapeDtypeStruct((_NC, _N, _D), jnp.float32),
        scratch_types=[
            pltpu.VMEM_SHARED((_N, _D), jnp.float32),
            pltpu.VMEM((2, _C), jnp.int32),
            pltpu.VMEM((2, _C), jnp.int32),
            pltpu.VMEM((2, _C), jnp.int32),
            pltpu.VMEM((_C,), jnp.float32),
            pltpu.VMEM((_C,), jnp.float32),
            pltpu.VMEM((_C,), jnp.float32),
            pltpu.VMEM((_C, _D), jnp.float32),
            pltpu.VMEM((_C, _D), jnp.float32),
            pltpu.VMEM((_C, _D), jnp.float32),
            pltpu.SemaphoreType.DMA,
            pltpu.SemaphoreType.DMA,
            pltpu.SemaphoreType.DMA,
        ],
    )
    return f(x, packed, ewp)


def _tc_tail_body(p_ref, w_ref, b_ref, a1_ref, g_ref, be_ref, a2_ref, o_ref):
    agg = p_ref[0] + p_ref[1]
    h = lax.dot_general(agg, w_ref[...], (((1,), (0,)), ((), ())),
                        preferred_element_type=jnp.float32,
                        precision=lax.Precision.HIGHEST)
    h = h + b_ref[...]
    a1 = a1_ref[0, 0]
    h = jnp.maximum(h, 0.0) + a1 * jnp.minimum(h, 0.0)
    mean = jnp.mean(h, axis=0, keepdims=True)
    var = jnp.mean((h - mean) ** 2, axis=0, keepdims=True)
    h = (h - mean) / jnp.sqrt(var + _EPS) * g_ref[...] + be_ref[...]
    a2 = a2_ref[0, 0]
    o_ref[...] = jnp.maximum(h, 0.0) + a2 * jnp.minimum(h, 0.0)


def _tc_tail(partials, W, b, a1, gamma, beta, a2):
    return pl.pallas_call(
        _tc_tail_body,
        out_shape=jax.ShapeDtypeStruct((_N, _D), jnp.float32),
        in_specs=[
            pl.BlockSpec(memory_space=pltpu.VMEM),
            pl.BlockSpec(memory_space=pltpu.VMEM),
            pl.BlockSpec(memory_space=pltpu.VMEM),
            pl.BlockSpec(memory_space=pltpu.SMEM),
            pl.BlockSpec(memory_space=pltpu.VMEM),
            pl.BlockSpec(memory_space=pltpu.VMEM),
            pl.BlockSpec(memory_space=pltpu.SMEM),
        ],
        out_specs=pl.BlockSpec(memory_space=pltpu.VMEM),
    )(partials, W, b.reshape(1, _D), a1.reshape(1, 1),
      gamma.reshape(1, _D), beta.reshape(1, _D), a2.reshape(1, 1))


def kernel(heat, edge_index, edge_weight, W, b, a1, gamma, beta, a2):
    src = edge_index[0]
    dst = edge_index[1]
    partials = _sc_aggregate(heat, src, dst, edge_weight)
    return _tc_tail(partials, W, b, a1, gamma, beta, a2)
